# baseline (speedup 1.0000x reference)
import functools

import jax
import jax.numpy as jnp
from jax import lax
from jax.experimental import pallas as pl
from jax.experimental.pallas import tpu as pltpu
from jax.experimental.pallas import tpu_sc as plsc

N_USER = 1000000
N_ITEM = 1000000
K = 16
BATCH = 16384

NC = 2
NS = 16
L = 16
NW = NC * NS
BPW = BATCH // NW
CHUNK = 128
NCHUNK = BPW // CHUNK
NBLK = BPW // L

_mesh = plsc.VectorSubcoreMesh(
    core_axis_name="c", subcore_axis_name="s", num_cores=NC, num_subcores=NS
)


@functools.partial(
    pl.kernel,
    out_type=jax.ShapeDtypeStruct((BATCH,), jnp.float32),
    mesh=_mesh,
    compiler_params=pltpu.CompilerParams(
        needs_layout_passes=False, use_tc_tiling_on_sc=False
    ),
    scratch_types=dict(
        uidx_v=pltpu.VMEM((NCHUNK, CHUNK), jnp.int32),
        iidx_v=pltpu.VMEM((NCHUNK, CHUNK), jnp.int32),
        u_rows=pltpu.VMEM((BPW, K), jnp.float32),
        v_rows=pltpu.VMEM((BPW, K), jnp.float32),
        bu_rows=pltpu.VMEM((BPW,), jnp.float32),
        bi_rows=pltpu.VMEM((BPW,), jnp.float32),
        bias_v=pltpu.VMEM((L,), jnp.float32),
        out_v=pltpu.VMEM((BPW,), jnp.float32),
        sem=pltpu.SemaphoreType.DMA,
    ),
)
def _mf_sc(
    uidx_hbm,
    iidx_hbm,
    user_emb,
    item_emb,
    bias_user,
    bias_item,
    bias16,
    out_hbm,
    *,
    uidx_v,
    iidx_v,
    u_rows,
    v_rows,
    bu_rows,
    bi_rows,
    bias_v,
    out_v,
    sem,
):
    wid = lax.axis_index("s") * NC + lax.axis_index("c")

    pltpu.sync_copy(uidx_hbm.at[wid], uidx_v)
    pltpu.sync_copy(iidx_hbm.at[wid], iidx_v)
    pltpu.sync_copy(bias16, bias_v)

    descs = []
    for c in range(NCHUNK):
        rows = pl.ds(c * CHUNK, CHUNK)
        descs.append(pltpu.async_copy(user_emb.at[uidx_v.at[c]], u_rows.at[rows], sem))
        descs.append(pltpu.async_copy(item_emb.at[iidx_v.at[c]], v_rows.at[rows], sem))
        descs.append(pltpu.async_copy(bias_user.at[uidx_v.at[c]], bu_rows.at[rows], sem))
        descs.append(pltpu.async_copy(bias_item.at[iidx_v.at[c]], bi_rows.at[rows], sem))
    for d in descs:
        d.wait()

    bias_vec = bias_v[...]
    lanes = lax.iota(jnp.int32, L)

    def blk(i, carry):
        b = pl.ds(i * L, L)
        r = i * L + lanes
        acc = bias_vec + bu_rows[b] + bi_rows[b]
        for k in range(K):
            kk = jnp.full((L,), k, jnp.int32)
            uk = plsc.load_gather(u_rows, [r, kk])
            vk = plsc.load_gather(v_rows, [r, kk])
            acc = acc + uk * vk
        out_v[b] = acc
        return carry

    lax.fori_loop(0, NBLK, blk, 0)

    pltpu.sync_copy(out_v, out_hbm.at[pl.ds(wid * BPW, BPW)])


def kernel(train_x, user_emb, item_emb, bias_user, bias_item, bias):
    uidx = train_x[:, 0].reshape(NW, NCHUNK, CHUNK).astype(jnp.int32)
    iidx = train_x[:, 1].reshape(NW, NCHUNK, CHUNK).astype(jnp.int32)
    bias16 = jnp.broadcast_to(bias.astype(jnp.float32), (L,))
    bu = bias_user.reshape(N_USER)
    bi = bias_item.reshape(N_ITEM)
    return _mf_sc(uidx, iidx, user_emb, item_emb, bu, bi, bias16)
